# baseline (device time: 53062 ns/iter reference)
import jax
import jax.numpy as jnp
from jax import lax
from jax.experimental import pallas as pl
from jax.experimental.pallas import tpu as pltpu

N_DEV = 4

X_ORDER = (1, 3, 2, 0)
W_SRC_OFF = (0, 1, 3, 2)


def kernel(x, w_mat, scale_x, scale_w):
    m_total, k_per = x.shape
    k_total, n_out = w_mat.shape
    m_per = m_total // N_DEV

    def mm(a, b):
        return lax.dot_general(
            a, b, (((1,), (0,)), ((), ())),
            preferred_element_type=jnp.float32,
        )

    def body(x_hbm, w_hbm, sx_ref, sw_ref, out_ref,
             xstage, x8, wstage, recv_ref,
             xcp_sems, wcp_sems, send_sems, recv_sems):
        my = lax.axis_index("i")

        def x_dma(i):
            t = lax.rem(my + X_ORDER[i], N_DEV)
            return pltpu.make_async_copy(
                x_hbm.at[pl.ds(t * m_per, m_per), :],
                xstage.at[i % 2],
                xcp_sems.at[i % 2],
            )

        def w_dma(j):
            src = lax.rem(my - W_SRC_OFF[j] + N_DEV, N_DEV)
            return pltpu.make_async_copy(
                w_hbm.at[pl.ds(src * k_per, k_per), :],
                wstage.at[j % 2],
                wcp_sems.at[j % 2],
            )

        def make_rdma(d, i):
            return pltpu.make_async_remote_copy(
                src_ref=x8.at[i],
                dst_ref=recv_ref.at[d - 1],
                send_sem=send_sems.at[d - 1],
                recv_sem=recv_sems.at[d - 1],
                device_id=(lax.rem(my + d, N_DEV),),
                device_id_type=pl.DeviceIdType.MESH,
            )

        x_dma(0).start()
        x_dma(1).start()

        barrier_sem = pltpu.get_barrier_semaphore()
        for d in range(1, N_DEV):
            pl.semaphore_signal(
                barrier_sem, inc=1,
                device_id=(lax.rem(my + d, N_DEV),),
                device_id_type=pl.DeviceIdType.MESH,
            )
        pl.semaphore_wait(barrier_sem, N_DEV - 1)

        rdmas = {}
        for i, d in enumerate(X_ORDER):
            x_dma(i).wait()
            x8[i, :, :] = xstage[i % 2].astype(jnp.float8_e4m3fn)
            if i == 0:
                x_dma(2).start()
                w_dma(0).start()
            elif i == 1:
                x_dma(3).start()
                w_dma(1).start()
            if d in (1, 3):
                rdmas[d] = make_rdma(d, i)
                rdmas[d].start()
        diag = make_rdma(2, 2)

        w_dma(0).wait()
        out_ref[...] = mm(x8[3], wstage[0][...].astype(jnp.float8_e5m2))
        w_dma(2).start()

        rdmas[1].wait_send()
        rdmas[3].wait_send()
        diag.start()

        w_dma(1).wait()
        rdmas[1].wait_recv()
        out_ref[...] += mm(recv_ref[0], wstage[1][...].astype(jnp.float8_e5m2))
        w_dma(3).start()

        w_dma(2).wait()
        rdmas[3].wait_recv()
        out_ref[...] += mm(recv_ref[2], wstage[0][...].astype(jnp.float8_e5m2))

        w_dma(3).wait()
        diag.wait_recv()
        out_ref[...] += mm(recv_ref[1], wstage[1][...].astype(jnp.float8_e5m2))

        diag.wait_send()

        out_ref[...] *= sx_ref[0] * sw_ref[0]

    return pl.pallas_call(
        body,
        out_shape=jax.ShapeDtypeStruct((m_per, n_out), jnp.float32),
        in_specs=[
            pl.BlockSpec(memory_space=pl.ANY),
            pl.BlockSpec(memory_space=pl.ANY),
            pl.BlockSpec(memory_space=pltpu.SMEM),
            pl.BlockSpec(memory_space=pltpu.SMEM),
        ],
        out_specs=pl.BlockSpec(memory_space=pltpu.VMEM),
        scratch_shapes=[
            pltpu.VMEM((2, m_per, k_per), jnp.float32),
            pltpu.VMEM((N_DEV, m_per, k_per),
                       jnp.float8_e4m3fn),
            pltpu.VMEM((2, k_per, n_out), jnp.float32),
            pltpu.VMEM((N_DEV - 1, m_per, k_per),
                       jnp.float8_e4m3fn),
            pltpu.SemaphoreType.DMA((2,)),
            pltpu.SemaphoreType.DMA((2,)),
            pltpu.SemaphoreType.DMA((N_DEV - 1,)),
            pltpu.SemaphoreType.DMA((N_DEV - 1,)),
        ],
        compiler_params=pltpu.CompilerParams(
            collective_id=0,
            vmem_limit_bytes=60 * 1024 * 1024,
        ),
    )(x, w_mat, scale_x, scale_w)
